# Initial kernel scaffold; baseline (speedup 1.0000x reference)
import jax
import jax.numpy as jnp
from jax.experimental import pallas as pl


def kernel(h_a_0, h_a_1, h_b_0, h_b_1, h_c_0, h_c_1, nbr_a_a, nbr_a_b, nbr_a_c, nbr_b_a, nbr_b_b, nbr_b_c, nbr_c_a, nbr_c_b, nbr_c_c, c_Wihf, c_Whhf, c_bihf, c_bhhf, c_Wihr, c_Whhr, c_bihr, c_bhhr, n_a_Wihf, n_a_Whhf, n_a_bihf, n_a_bhhf, n_a_Wihr, n_a_Whhr, n_a_bihr, n_a_bhhr, n_b_Wihf, n_b_Whhf, n_b_bihf, n_b_bhhf, n_b_Wihr, n_b_Whhr, n_b_bihr, n_b_bhhr, n_c_Wihf, n_c_Whhf, n_c_bihf, n_c_bhhf, n_c_Wihr, n_c_Whhr, n_c_bihr, n_c_bhhr, att_a_W, att_a_b, att_b_W, att_b_b, att_c_W, att_c_b):
    raise NotImplementedError("write your pallas kernel here")


# speedup vs baseline: 4.3734x; 4.3734x over previous
import functools

import jax
import jax.numpy as jnp
from jax import lax
from jax.experimental import pallas as pl
from jax.experimental.pallas import tpu as pltpu
from jax.experimental.pallas import tpu_sc as plsc

_N = 4096
_K = 8
_D = 128
_H = 64
_G = 4 * _H
_NE = 9
_NW = 32
_BA = 512
_BB = 512
_CH = 512
_ROWS = _NE * _K * _N
_PW = _ROWS // _NW

_pc = pl.pallas_call


def _dot_t(x, w):
    return lax.dot_general(x, w, (((1,), (1,)), ((), ())),
                           preferred_element_type=jnp.float32)


def _cell(g, c_prev):
    i = jax.nn.sigmoid(g[:, :_H])
    f = jax.nn.sigmoid(g[:, _H:2 * _H])
    gg = jnp.tanh(g[:, 2 * _H:3 * _H])
    o = jax.nn.sigmoid(g[:, 3 * _H:])
    c = f * c_prev + i * gg
    return o * jnp.tanh(c), c


def _cell0(g):
    i = jax.nn.sigmoid(g[:, :_H])
    gg = jnp.tanh(g[:, 2 * _H:3 * _H])
    o = jax.nn.sigmoid(g[:, 3 * _H:])
    c = i * gg
    return o * jnp.tanh(c), c


def _content_body(x0a, x1a, x0b, x1b, x0c, x1c,
                  wf_r, whf_r, bf_r, wr_r, whr_r, br_r, out_r):
    wf = wf_r[...]
    whf = whf_r[...]
    bf = bf_r[...]
    wr = wr_r[...]
    whr = whr_r[...]
    br = br_r[...]
    for n, (x0r, x1r) in enumerate(((x0a, x1a), (x0b, x1b), (x0c, x1c))):
        x0 = x0r[...]
        x1 = x1r[...]
        h1, c1 = _cell0(_dot_t(x0, wf) + bf)
        h2, _ = _cell(_dot_t(x1, wf) + _dot_t(h1, whf) + bf, c1)
        b1, cb1 = _cell0(_dot_t(x1, wr) + br)
        b2, _ = _cell(_dot_t(x0, wr) + _dot_t(b1, whr) + br, cb1)
        out_r[n] = jnp.concatenate([0.5 * (h1 + h2), 0.5 * (b1 + b2)], axis=1)


def _content_call(h_pairs, wf, whf, bf, wr, whr, br):
    full = lambda s: pl.BlockSpec(s, lambda i: tuple(0 for _ in s))
    return _pc(
        _content_body,
        grid=(_N // _BA,),
        in_specs=[pl.BlockSpec((_BA, _D), lambda i: (i, 0))] * 6 + [
            full((_G, _D)), full((_G, _H)), full((1, _G)),
            full((_G, _D)), full((_G, _H)), full((1, _G)),
        ],
        out_specs=pl.BlockSpec((3, _BA, _D), lambda i: (0, i, 0)),
        out_shape=jax.ShapeDtypeStruct((3, _N, _D), jnp.float32),
    )(*h_pairs, wf, whf, bf, wr, whr, br)


def _sc_gather(table, idx):
    mesh = plsc.VectorSubcoreMesh(core_axis_name="c", subcore_axis_name="s")

    @functools.partial(
        pl.kernel, mesh=mesh,
        out_type=jax.ShapeDtypeStruct((_ROWS, _D), jnp.float32),
        scratch_types=[
            pltpu.VMEM((_CH,), jnp.int32),
            pltpu.VMEM((_CH, _D), jnp.float32),
            pltpu.SemaphoreType.DMA,
        ],
    )
    def gk(table_hbm, idx_hbm, out_hbm, idx_v, rows_v, sem):
        wid = lax.axis_index("s") * 2 + lax.axis_index("c")
        base = wid * _PW

        def body(ci, carry):
            off = base + ci * _CH
            pltpu.sync_copy(idx_hbm.at[pl.ds(off, _CH)], idx_v)
            pltpu.async_copy(table_hbm.at[idx_v], rows_v, sem).wait()
            pltpu.sync_copy(rows_v, out_hbm.at[pl.ds(off, _CH)])
            return carry

        lax.fori_loop(0, _PW // _CH, body, 0)

    return gk(table, idx)


def _neigh_body(m_r, wf_r, whf_r, bf_r, wr_r, whr_r, br_r, out_r):
    m = m_r[...].reshape(_K * _BB, _D)
    wf = wf_r[0]
    whf = whf_r[0]
    bf = bf_r[0]
    wr = wr_r[0]
    whr = whr_r[0]
    br = br_r[0]
    gxf = _dot_t(m, wf) + bf
    gxr = _dot_t(m, wr) + br

    h, c = _cell0(gxf[:_BB])
    accf = h
    for t in range(1, _K):
        h, c = _cell(gxf[t * _BB:(t + 1) * _BB] + _dot_t(h, whf), c)
        accf += h

    h, c = _cell0(gxr[(_K - 1) * _BB:])
    accr = h
    for t in range(_K - 2, -1, -1):
        h, c = _cell(gxr[t * _BB:(t + 1) * _BB] + _dot_t(h, whr), c)
        accr += h

    out_r[0] = jnp.concatenate([accf, accr], axis=1) * (1.0 / _K)


def _neigh_call(mailbox, wf, whf, bf, wr, whr, br):
    wspec = lambda s: pl.BlockSpec(s, lambda e, b: (e // 3, 0, 0))
    return _pc(
        _neigh_body,
        grid=(_NE, _N // _BB),
        in_specs=[
            pl.BlockSpec((1, _K, _BB, _D), lambda e, b: (e, 0, b, 0)),
            wspec((1, _G, _D)), wspec((1, _G, _H)), wspec((1, 1, _G)),
            wspec((1, _G, _D)), wspec((1, _G, _H)), wspec((1, 1, _G)),
        ],
        out_specs=pl.BlockSpec((1, _BB, _D), lambda e, b: (e, b, 0)),
        out_shape=jax.ShapeDtypeStruct((_NE, _N, _D), jnp.float32),
    )(mailbox, wf, whf, bf, wr, whr, br)


def _att_body(n0_r, n1_r, n2_r, c_r, w_r, b_r, out_r):
    dh = c_r[0]
    e0 = n0_r[0]
    e1 = n1_r[0]
    e2 = n2_r[0]
    w = w_r[0]
    b = b_r[0][:, :1]
    w1 = w[:, :_D]
    w2 = w[:, _D:]
    sd = jnp.sum(dh * w1, axis=1, keepdims=True)

    def score(e):
        s = sd + jnp.sum(e * w2, axis=1, keepdims=True) + b
        return jnp.where(s >= 0, s, 0.01 * s)

    s0, s1, s2, s3 = score(e0), score(e1), score(e2), score(dh)
    mx = jnp.maximum(jnp.maximum(s0, s1), jnp.maximum(s2, s3))
    p0 = jnp.exp(s0 - mx)
    p1 = jnp.exp(s1 - mx)
    p2 = jnp.exp(s2 - mx)
    p3 = jnp.exp(s3 - mx)
    z = p0 + p1 + p2 + p3
    out_r[0] = (e0 * p0 + e1 * p1 + e2 * p2 + dh * p3) / z


def _att_call(neigh, content, attw, attb):
    blk = lambda f: pl.BlockSpec((1, _BB, _D), f)
    return _pc(
        _att_body,
        grid=(3, _N // _BB),
        in_specs=[
            blk(lambda n, b: (n, b, 0)),
            blk(lambda n, b: (n + 3, b, 0)),
            blk(lambda n, b: (n + 6, b, 0)),
            blk(lambda n, b: (n, b, 0)),
            pl.BlockSpec((1, 1, 2 * _D), lambda n, b: (n, 0, 0)),
            pl.BlockSpec((1, 1, _D), lambda n, b: (n, 0, 0)),
        ],
        out_specs=pl.BlockSpec((1, _BB, _D), lambda n, b: (n, b, 0)),
        out_shape=jax.ShapeDtypeStruct((3, _N, _D), jnp.float32),
    )(neigh, neigh, neigh, content, attw, attb)


def kernel(h_a_0, h_a_1, h_b_0, h_b_1, h_c_0, h_c_1,
           nbr_a_a, nbr_a_b, nbr_a_c, nbr_b_a, nbr_b_b, nbr_b_c,
           nbr_c_a, nbr_c_b, nbr_c_c,
           c_Wihf, c_Whhf, c_bihf, c_bhhf, c_Wihr, c_Whhr, c_bihr, c_bhhr,
           n_a_Wihf, n_a_Whhf, n_a_bihf, n_a_bhhf,
           n_a_Wihr, n_a_Whhr, n_a_bihr, n_a_bhhr,
           n_b_Wihf, n_b_Whhf, n_b_bihf, n_b_bhhf,
           n_b_Wihr, n_b_Whhr, n_b_bihr, n_b_bhhr,
           n_c_Wihf, n_c_Whhf, n_c_bihf, n_c_bhhf,
           n_c_Wihr, n_c_Whhr, n_c_bihr, n_c_bhhr,
           att_a_W, att_a_b, att_b_W, att_b_b, att_c_W, att_c_b):
    bf_c = (c_bihf + c_bhhf).reshape(1, _G)
    br_c = (c_bihr + c_bhhr).reshape(1, _G)
    content = _content_call(
        (h_a_0, h_a_1, h_b_0, h_b_1, h_c_0, h_c_1),
        c_Wihf, c_Whhf, bf_c, c_Wihr, c_Whhr, br_c)

    nbrs = (nbr_a_a, nbr_a_b, nbr_a_c, nbr_b_a, nbr_b_b, nbr_b_c,
            nbr_c_a, nbr_c_b, nbr_c_c)
    idx = jnp.concatenate(
        [(nbrs[e].T + (e // 3) * _N).reshape(_K * _N) for e in range(_NE)])
    mailbox = _sc_gather(content.reshape(3 * _N, _D), idx)
    mailbox = mailbox.reshape(_NE, _K, _N, _D)

    wf = jnp.stack([n_a_Wihf, n_b_Wihf, n_c_Wihf])
    whf = jnp.stack([n_a_Whhf, n_b_Whhf, n_c_Whhf])
    bf = jnp.stack([n_a_bihf + n_a_bhhf, n_b_bihf + n_b_bhhf,
                    n_c_bihf + n_c_bhhf]).reshape(3, 1, _G)
    wr = jnp.stack([n_a_Wihr, n_b_Wihr, n_c_Wihr])
    whr = jnp.stack([n_a_Whhr, n_b_Whhr, n_c_Whhr])
    br = jnp.stack([n_a_bihr + n_a_bhhr, n_b_bihr + n_b_bhhr,
                    n_c_bihr + n_c_bhhr]).reshape(3, 1, _G)
    neigh = _neigh_call(mailbox, wf, whf, bf, wr, whr, br)

    attw = jnp.stack([att_a_W, att_b_W, att_c_W])
    attb = jnp.broadcast_to(
        jnp.stack([att_a_b, att_b_b, att_c_b]).reshape(3, 1, 1), (3, 1, _D))
    return _att_call(neigh, content, attw, attb)


# speedup vs baseline: 5.5519x; 1.2695x over previous
import functools

import jax
import jax.numpy as jnp
from jax import lax
from jax.experimental import pallas as pl
from jax.experimental.pallas import tpu as pltpu
from jax.experimental.pallas import tpu_sc as plsc

_N = 4096
_K = 8
_D = 128
_H = 64
_G = 4 * _H
_NW = 32
_BA = 512
_BB = 512
_CH = 512
_ROWS = 3 * _K * _N
_PW = _ROWS // _NW

_pc = pl.pallas_call


def _dot_t(x, w):
    return lax.dot_general(x, w, (((1,), (1,)), ((), ())),
                           preferred_element_type=jnp.float32)


def _sig(x):
    return 0.5 + 0.5 * jnp.tanh(0.5 * x)


def _cell(g, c_prev):
    i = _sig(g[:, :_H])
    f = _sig(g[:, _H:2 * _H])
    gg = jnp.tanh(g[:, 2 * _H:3 * _H])
    o = _sig(g[:, 3 * _H:])
    c = f * c_prev + i * gg
    return o * jnp.tanh(c), c


def _cell0(g):
    i = _sig(g[:, :_H])
    gg = jnp.tanh(g[:, 2 * _H:3 * _H])
    o = _sig(g[:, 3 * _H:])
    c = i * gg
    return o * jnp.tanh(c), c


def _content_body(x0a, x1a, x0b, x1b, x0c, x1c,
                  wf_r, whf_r, bf_r, wr_r, whr_r, br_r, out_r):
    wf = wf_r[...]
    whf = whf_r[...]
    bf = bf_r[...]
    wr = wr_r[...]
    whr = whr_r[...]
    br = br_r[...]
    for n, (x0r, x1r) in enumerate(((x0a, x1a), (x0b, x1b), (x0c, x1c))):
        x0 = x0r[...]
        x1 = x1r[...]
        h1, c1 = _cell0(_dot_t(x0, wf) + bf)
        h2, _ = _cell(_dot_t(x1, wf) + _dot_t(h1, whf) + bf, c1)
        b1, cb1 = _cell0(_dot_t(x1, wr) + br)
        b2, _ = _cell(_dot_t(x0, wr) + _dot_t(b1, whr) + br, cb1)
        out_r[n] = jnp.concatenate([0.5 * (h1 + h2), 0.5 * (b1 + b2)], axis=1)


def _content_call(h_pairs, wf, whf, bf, wr, whr, br):
    full = lambda s: pl.BlockSpec(s, lambda i: tuple(0 for _ in s))
    return _pc(
        _content_body,
        grid=(_N // _BA,),
        in_specs=[pl.BlockSpec((_BA, _D), lambda i: (i, 0))] * 6 + [
            full((_G, _D)), full((_G, _H)), full((1, _G)),
            full((_G, _D)), full((_G, _H)), full((1, _G)),
        ],
        out_specs=pl.BlockSpec((3, _BA, _D), lambda i: (0, i, 0)),
        out_shape=jax.ShapeDtypeStruct((3, _N, _D), jnp.float32),
    )(*h_pairs, wf, whf, bf, wr, whr, br)


def _sc_gather(table, idx):
    mesh = plsc.VectorSubcoreMesh(core_axis_name="c", subcore_axis_name="s")

    @functools.partial(
        pl.kernel, mesh=mesh,
        out_type=jax.ShapeDtypeStruct((_ROWS, _D), jnp.float32),
        scratch_types=[
            pltpu.VMEM((_CH,), jnp.int32),
            pltpu.VMEM((_CH, _D), jnp.float32),
            pltpu.SemaphoreType.DMA,
        ],
    )
    def gk(table_hbm, idx_hbm, out_hbm, idx_v, rows_v, sem):
        wid = lax.axis_index("s") * 2 + lax.axis_index("c")
        base = wid * _PW

        def body(ci, carry):
            off = base + ci * _CH
            pltpu.sync_copy(idx_hbm.at[pl.ds(off, _CH)], idx_v)
            pltpu.async_copy(table_hbm.at[idx_v], rows_v, sem).wait()
            pltpu.sync_copy(rows_v, out_hbm.at[pl.ds(off, _CH)])
            return carry

        lax.fori_loop(0, _PW // _CH, body, 0)

    return gk(table, idx)


def _neigh_body(m_r, wf_r, whf_r, bf_r, wr_r, whr_r, br_r, out_r):
    m = m_r[...].reshape(_K * _BB, _D)
    wf = wf_r[...]
    whf = whf_r[...]
    bf = bf_r[...]
    wr = wr_r[...]
    whr = whr_r[...]
    br = br_r[...]
    gxf = _dot_t(m, wf) + bf
    gxr = _dot_t(m, wr) + br

    h, c = _cell0(gxf[:_BB])
    accf = h
    for t in range(1, _K):
        h, c = _cell(gxf[t * _BB:(t + 1) * _BB] + _dot_t(h, whf), c)
        accf += h

    h, c = _cell0(gxr[(_K - 1) * _BB:])
    accr = h
    for t in range(_K - 2, -1, -1):
        h, c = _cell(gxr[t * _BB:(t + 1) * _BB] + _dot_t(h, whr), c)
        accr += h

    out_r[0] = jnp.concatenate([accf, accr], axis=1) * (1.0 / _K)


def _neigh_call(mailbox, wf, whf, bf, wr, whr, br):
    full = lambda s: pl.BlockSpec(s, lambda d, b: tuple(0 for _ in s))
    return _pc(
        _neigh_body,
        grid=(3, _N // _BB),
        in_specs=[
            pl.BlockSpec((1, _K, _BB, _D), lambda d, b: (d, 0, b, 0)),
            full((_G, _D)), full((_G, _H)), full((1, _G)),
            full((_G, _D)), full((_G, _H)), full((1, _G)),
        ],
        out_specs=pl.BlockSpec((1, _BB, _D), lambda d, b: (d, b, 0)),
        out_shape=jax.ShapeDtypeStruct((3, _N, _D), jnp.float32),
    )(mailbox, wf, whf, bf, wr, whr, br)


def _att_body(n0_r, n1_r, n2_r, c_r, w_r, b_r, out_r):
    dh = c_r[0]
    e0 = n0_r[0]
    e1 = n1_r[0]
    e2 = n2_r[0]
    w = w_r[0]
    b = b_r[0][:, :1]
    w1 = w[:, :_D]
    w2 = w[:, _D:]
    sd = jnp.sum(dh * w1, axis=1, keepdims=True)

    def score(e):
        s = sd + jnp.sum(e * w2, axis=1, keepdims=True) + b
        return jnp.where(s >= 0, s, 0.01 * s)

    s0, s1, s2, s3 = score(e0), score(e1), score(e2), score(dh)
    mx = jnp.maximum(jnp.maximum(s0, s1), jnp.maximum(s2, s3))
    p0 = jnp.exp(s0 - mx)
    p1 = jnp.exp(s1 - mx)
    p2 = jnp.exp(s2 - mx)
    p3 = jnp.exp(s3 - mx)
    z = p0 + p1 + p2 + p3
    out_r[0] = (e0 * p0 + e1 * p1 + e2 * p2 + dh * p3) / z


def _att_call(neigh_a, neigh_b, neigh_c, content, attw, attb):
    blk = pl.BlockSpec((1, _BB, _D), lambda n, b: (n, b, 0))
    return _pc(
        _att_body,
        grid=(3, _N // _BB),
        in_specs=[
            blk, blk, blk, blk,
            pl.BlockSpec((1, 1, 2 * _D), lambda n, b: (n, 0, 0)),
            pl.BlockSpec((1, 1, _D), lambda n, b: (n, 0, 0)),
        ],
        out_specs=pl.BlockSpec((1, _BB, _D), lambda n, b: (n, b, 0)),
        out_shape=jax.ShapeDtypeStruct((3, _N, _D), jnp.float32),
    )(neigh_a, neigh_b, neigh_c, content, attw, attb)


def kernel(h_a_0, h_a_1, h_b_0, h_b_1, h_c_0, h_c_1,
           nbr_a_a, nbr_a_b, nbr_a_c, nbr_b_a, nbr_b_b, nbr_b_c,
           nbr_c_a, nbr_c_b, nbr_c_c,
           c_Wihf, c_Whhf, c_bihf, c_bhhf, c_Wihr, c_Whhr, c_bihr, c_bhhr,
           n_a_Wihf, n_a_Whhf, n_a_bihf, n_a_bhhf,
           n_a_Wihr, n_a_Whhr, n_a_bihr, n_a_bhhr,
           n_b_Wihf, n_b_Whhf, n_b_bihf, n_b_bhhf,
           n_b_Wihr, n_b_Whhr, n_b_bihr, n_b_bhhr,
           n_c_Wihf, n_c_Whhf, n_c_bihf, n_c_bhhf,
           n_c_Wihr, n_c_Whhr, n_c_bihr, n_c_bhhr,
           att_a_W, att_a_b, att_b_W, att_b_b, att_c_W, att_c_b):
    bf_c = (c_bihf + c_bhhf).reshape(1, _G)
    br_c = (c_bihr + c_bhhr).reshape(1, _G)
    content = _content_call(
        (h_a_0, h_a_1, h_b_0, h_b_1, h_c_0, h_c_1),
        c_Wihf, c_Whhf, bf_c, c_Wihr, c_Whhr, br_c)
    table = content.reshape(3 * _N, _D)

    nbrs = ((nbr_a_a, nbr_a_b, nbr_a_c),
            (nbr_b_a, nbr_b_b, nbr_b_c),
            (nbr_c_a, nbr_c_b, nbr_c_c))
    nweights = ((n_a_Wihf, n_a_Whhf, n_a_bihf, n_a_bhhf,
                 n_a_Wihr, n_a_Whhr, n_a_bihr, n_a_bhhr),
                (n_b_Wihf, n_b_Whhf, n_b_bihf, n_b_bhhf,
                 n_b_Wihr, n_b_Whhr, n_b_bihr, n_b_bhhr),
                (n_c_Wihf, n_c_Whhf, n_c_bihf, n_c_bhhf,
                 n_c_Wihr, n_c_Whhr, n_c_bihr, n_c_bhhr))

    neigh = []
    for s in range(3):
        idx = jnp.concatenate(
            [(nbrs[s][d].T + s * _N).reshape(_K * _N) for d in range(3)])
        mb = _sc_gather(table, idx).reshape(3, _K, _N, _D)
        wihf, whhf, bihf, bhhf, wihr, whhr, bihr, bhhr = nweights[s]
        neigh.append(_neigh_call(
            mb, wihf, whhf, (bihf + bhhf).reshape(1, _G),
            wihr, whhr, (bihr + bhhr).reshape(1, _G)))

    attw = jnp.stack([att_a_W, att_b_W, att_c_W])
    attb = jnp.broadcast_to(
        jnp.stack([att_a_b, att_b_b, att_c_b]).reshape(3, 1, 1), (3, 1, _D))
    return _att_call(neigh[0], neigh[1], neigh[2], content, attw, attb)


# speedup vs baseline: 5.7590x; 1.0373x over previous
import functools

import jax
import jax.numpy as jnp
from jax import lax
from jax.experimental import pallas as pl
from jax.experimental.pallas import tpu as pltpu
from jax.experimental.pallas import tpu_sc as plsc

_N = 4096
_K = 8
_D = 128
_H = 64
_G = 4 * _H
_NW = 32
_BA = 512
_BB = 512
_CH = 512
_ROWS = _K * _N
_PW = _ROWS // _NW

_pc = pl.pallas_call

def _prescale(w):
    s = jnp.concatenate([jnp.full((2 * _H,), 0.5, jnp.float32),
                         jnp.ones((_H,), jnp.float32),
                         jnp.full((_H,), 0.5, jnp.float32)])
    return w * s.reshape((_G,) + (1,) * (w.ndim - 1))


def _dot_t(x, w):
    return lax.dot_general(x, w, (((1,), (1,)), ((), ())),
                           preferred_element_type=jnp.float32)


def _cell(gs, c_prev):
    t = jnp.tanh(gs)
    u = 0.5 * t + 0.5
    c = u[:, _H:2 * _H] * c_prev + u[:, :_H] * t[:, 2 * _H:3 * _H]
    return u[:, 3 * _H:] * jnp.tanh(c), c


def _cell0(gs):
    t = jnp.tanh(gs)
    u = 0.5 * t + 0.5
    c = u[:, :_H] * t[:, 2 * _H:3 * _H]
    return u[:, 3 * _H:] * jnp.tanh(c), c


def _content_body(x0a, x1a, x0b, x1b, x0c, x1c,
                  wf_r, whf_r, bf_r, wr_r, whr_r, br_r, out_r):
    wf = wf_r[...]
    whf = whf_r[...]
    bf = bf_r[...]
    wr = wr_r[...]
    whr = whr_r[...]
    br = br_r[...]
    for n, (x0r, x1r) in enumerate(((x0a, x1a), (x0b, x1b), (x0c, x1c))):
        x0 = x0r[...]
        x1 = x1r[...]
        h1, c1 = _cell0(_dot_t(x0, wf) + bf)
        h2, _ = _cell(_dot_t(x1, wf) + _dot_t(h1, whf) + bf, c1)
        b1, cb1 = _cell0(_dot_t(x1, wr) + br)
        b2, _ = _cell(_dot_t(x0, wr) + _dot_t(b1, whr) + br, cb1)
        out_r[n] = jnp.concatenate([0.5 * (h1 + h2), 0.5 * (b1 + b2)], axis=1)


def _content_call(h_pairs, wf, whf, bf, wr, whr, br):
    full = lambda s: pl.BlockSpec(s, lambda i: tuple(0 for _ in s))
    return _pc(
        _content_body,
        grid=(_N // _BA,),
        in_specs=[pl.BlockSpec((_BA, _D), lambda i: (i, 0))] * 6 + [
            full((_G, _D)), full((_G, _H)), full((1, _G)),
            full((_G, _D)), full((_G, _H)), full((1, _G)),
        ],
        out_specs=pl.BlockSpec((3, _BA, _D), lambda i: (0, i, 0)),
        out_shape=jax.ShapeDtypeStruct((3, _N, _D), jnp.float32),
    )(*h_pairs, wf, whf, bf, wr, whr, br)


def _sc_gather(table, idx):
    mesh = plsc.VectorSubcoreMesh(core_axis_name="c", subcore_axis_name="s")

    @functools.partial(
        pl.kernel, mesh=mesh,
        out_type=jax.ShapeDtypeStruct((_ROWS, _D), jnp.float32),
        scratch_types=[
            pltpu.VMEM((_CH,), jnp.int32),
            pltpu.VMEM((_CH, _D), jnp.float32),
            pltpu.SemaphoreType.DMA,
        ],
    )
    def gk(table_hbm, idx_hbm, out_hbm, idx_v, rows_v, sem):
        wid = lax.axis_index("s") * 2 + lax.axis_index("c")
        base = wid * _PW

        def body(ci, carry):
            off = base + ci * _CH
            pltpu.sync_copy(idx_hbm.at[pl.ds(off, _CH)], idx_v)
            pltpu.async_copy(table_hbm.at[idx_v], rows_v, sem).wait()
            pltpu.sync_copy(rows_v, out_hbm.at[pl.ds(off, _CH)])
            return carry

        lax.fori_loop(0, _PW // _CH, body, 0)

    return gk(table, idx)


def _neigh_body(m_r, wf_r, whf_r, bf_r, wr_r, whr_r, br_r, out_r):
    m = m_r[...].reshape(_K * _BB, _D)
    wf = wf_r[...]
    whf = whf_r[...]
    bf = bf_r[...]
    wr = wr_r[...]
    whr = whr_r[...]
    br = br_r[...]
    gxf = _dot_t(m, wf) + bf
    gxr = _dot_t(m, wr) + br

    h, c = _cell0(gxf[:_BB])
    accf = h
    for t in range(1, _K):
        h, c = _cell(gxf[t * _BB:(t + 1) * _BB] + _dot_t(h, whf), c)
        accf += h

    h, c = _cell0(gxr[(_K - 1) * _BB:])
    accr = h
    for t in range(_K - 2, -1, -1):
        h, c = _cell(gxr[t * _BB:(t + 1) * _BB] + _dot_t(h, whr), c)
        accr += h

    out_r[...] = jnp.concatenate([accf, accr], axis=1) * (1.0 / _K)


def _neigh_call(mailbox, wf, whf, bf, wr, whr, br):
    full = lambda s: pl.BlockSpec(s, lambda b: tuple(0 for _ in s))
    return _pc(
        _neigh_body,
        grid=(_N // _BB,),
        in_specs=[
            pl.BlockSpec((_K, _BB, _D), lambda b: (0, b, 0)),
            full((_G, _D)), full((_G, _H)), full((1, _G)),
            full((_G, _D)), full((_G, _H)), full((1, _G)),
        ],
        out_specs=pl.BlockSpec((_BB, _D), lambda b: (b, 0)),
        out_shape=jax.ShapeDtypeStruct((_N, _D), jnp.float32),
    )(mailbox, wf, whf, bf, wr, whr, br)


def _att_body(*refs):
    n_refs = refs[:9]
    c_r, w_r, b_r, out_r = refs[9:]
    for n in range(3):
        dh = c_r[n]
        e0 = n_refs[n][...]
        e1 = n_refs[3 + n][...]
        e2 = n_refs[6 + n][...]
        w = w_r[n]
        b = b_r[n][:, :1]
        w1 = w[:, :_D]
        w2 = w[:, _D:]
        sd = jnp.sum(dh * w1, axis=1, keepdims=True)

        def score(e):
            s = sd + jnp.sum(e * w2, axis=1, keepdims=True) + b
            return jnp.where(s >= 0, s, 0.01 * s)

        s0, s1, s2, s3 = score(e0), score(e1), score(e2), score(dh)
        mx = jnp.maximum(jnp.maximum(s0, s1), jnp.maximum(s2, s3))
        p0 = jnp.exp(s0 - mx)
        p1 = jnp.exp(s1 - mx)
        p2 = jnp.exp(s2 - mx)
        p3 = jnp.exp(s3 - mx)
        z = p0 + p1 + p2 + p3
        out_r[n] = (e0 * p0 + e1 * p1 + e2 * p2 + dh * p3) / z


def _att_call(neighs, content, attw, attb):
    blk = pl.BlockSpec((_BB, _D), lambda b: (b, 0))
    return _pc(
        _att_body,
        grid=(_N // _BB,),
        in_specs=[blk] * 9 + [
            pl.BlockSpec((3, _BB, _D), lambda b: (0, b, 0)),
            pl.BlockSpec((3, 1, 2 * _D), lambda b: (0, 0, 0)),
            pl.BlockSpec((3, 1, _D), lambda b: (0, 0, 0)),
        ],
        out_specs=pl.BlockSpec((3, _BB, _D), lambda b: (0, b, 0)),
        out_shape=jax.ShapeDtypeStruct((3, _N, _D), jnp.float32),
    )(*neighs, content, attw, attb)


def kernel(h_a_0, h_a_1, h_b_0, h_b_1, h_c_0, h_c_1,
           nbr_a_a, nbr_a_b, nbr_a_c, nbr_b_a, nbr_b_b, nbr_b_c,
           nbr_c_a, nbr_c_b, nbr_c_c,
           c_Wihf, c_Whhf, c_bihf, c_bhhf, c_Wihr, c_Whhr, c_bihr, c_bhhr,
           n_a_Wihf, n_a_Whhf, n_a_bihf, n_a_bhhf,
           n_a_Wihr, n_a_Whhr, n_a_bihr, n_a_bhhr,
           n_b_Wihf, n_b_Whhf, n_b_bihf, n_b_bhhf,
           n_b_Wihr, n_b_Whhr, n_b_bihr, n_b_bhhr,
           n_c_Wihf, n_c_Whhf, n_c_bihf, n_c_bhhf,
           n_c_Wihr, n_c_Whhr, n_c_bihr, n_c_bhhr,
           att_a_W, att_a_b, att_b_W, att_b_b, att_c_W, att_c_b):
    content = _content_call(
        (h_a_0, h_a_1, h_b_0, h_b_1, h_c_0, h_c_1),
        _prescale(c_Wihf), _prescale(c_Whhf),
        _prescale(c_bihf + c_bhhf).reshape(1, _G),
        _prescale(c_Wihr), _prescale(c_Whhr),
        _prescale(c_bihr + c_bhhr).reshape(1, _G))
    table = content.reshape(3 * _N, _D)

    nbrs = (nbr_a_a, nbr_a_b, nbr_a_c, nbr_b_a, nbr_b_b, nbr_b_c,
            nbr_c_a, nbr_c_b, nbr_c_c)
    nweights = ((n_a_Wihf, n_a_Whhf, n_a_bihf, n_a_bhhf,
                 n_a_Wihr, n_a_Whhr, n_a_bihr, n_a_bhhr),
                (n_b_Wihf, n_b_Whhf, n_b_bihf, n_b_bhhf,
                 n_b_Wihr, n_b_Whhr, n_b_bihr, n_b_bhhr),
                (n_c_Wihf, n_c_Whhf, n_c_bihf, n_c_bhhf,
                 n_c_Wihr, n_c_Whhr, n_c_bihr, n_c_bhhr))

    neighs = []
    for e in range(9):
        s = e // 3
        idx = (nbrs[e].T + s * _N).reshape(_ROWS)
        mb = _sc_gather(table, idx).reshape(_K, _N, _D)
        wihf, whhf, bihf, bhhf, wihr, whhr, bihr, bhhr = nweights[s]
        neighs.append(_neigh_call(
            mb, _prescale(wihf), _prescale(whhf),
            _prescale(bihf + bhhf).reshape(1, _G),
            _prescale(wihr), _prescale(whhr),
            _prescale(bihr + bhhr).reshape(1, _G)))

    attw = jnp.stack([att_a_W, att_b_W, att_c_W])
    attb = jnp.broadcast_to(
        jnp.stack([att_a_b, att_b_b, att_c_b]).reshape(3, 1, 1), (3, 1, _D))
    return _att_call(neighs, content, attw, attb)
